# Initial kernel scaffold; baseline (speedup 1.0000x reference)
import jax
import jax.numpy as jnp
from jax.experimental import pallas as pl


def kernel(x, edge_index, edge_attr, angle_edge_index, angle_attr, is_reactant, Wa, ba, Wb, bb, gamma, beta, W1, b1, W2, b2):
    raise NotImplementedError("write your pallas kernel here")


# speedup vs baseline: 3.7306x; 3.7306x over previous
import functools

import jax
import jax.numpy as jnp
from jax import lax
from jax.experimental import pallas as pl
from jax.experimental.pallas import tpu as pltpu
from jax.experimental.pallas import tpu_sc as plsc

_N = 10000
_E = 320000
_A = 640000
_D = 32
_B = 200
_NPG = 50
_H = 128
_L = 3

_NCHUNK = 16
_CH = 20480
_EPAD = _NCHUNK * _CH
_DUM = 64
_WG = 512
_NW_A = _A // _WG
_NW_E = _E // _WG

_mesh = plsc.VectorSubcoreMesh(core_axis_name="c", subcore_axis_name="s")


def _zero_vmem(ref, rows):
    z = jnp.zeros((16,), jnp.float32)

    def body(i, _):
        ref[i, pl.ds(0, 16)] = z
        ref[i, pl.ds(16, 16)] = z
        return 0

    lax.fori_loop(0, rows, body, 0)


def _localize_dst(didx_v, lo):
    lane = lax.iota(jnp.int32, 16)

    def body(i, _):
        dv = didx_v[pl.ds(i * 16, 16)]
        dl = dv - lo
        oob = (dl < 0) | (dl >= _CH)
        dummy = _CH + ((dv + lane) & (_DUM - 1))
        didx_v[pl.ds(i * 16, 16)] = jnp.where(oob, dummy, dl)
        return 0

    lax.fori_loop(0, _WG // 16, body, 0)


def _chunked_scatter_body(gather_rows):
    def make(sidx_hbm, didx_hbm, rows_hbm, init_hbm, out_hbm,
             sidx_v, didx_v, rows_v, st_v, acc_sh, sem):
        c = lax.axis_index("c")
        s = lax.axis_index("s")
        n_w = jnp.where(s < _NW_A % 16, _NW_A // 16 + 1, _NW_A // 16)
        tpr = _CH // 16
        if not gather_rows:
            _zero_vmem(st_v, tpr)

        def one_pass(p, _):
            lo = (c * (_NCHUNK // 2) + p) * _CH
            plsc.subcore_barrier()
            if gather_rows:
                pltpu.sync_copy(init_hbm.at[pl.ds(lo + s * tpr, tpr)], st_v)
            pltpu.sync_copy(st_v, acc_sh.at[pl.ds(s * tpr, tpr)])

            @pl.when(s == 0)
            def _():
                if gather_rows:
                    _zero_vmem(rows_v, _DUM)
                    pltpu.sync_copy(rows_v.at[pl.ds(0, _DUM)],
                                    acc_sh.at[pl.ds(_CH, _DUM)])
                else:
                    pltpu.sync_copy(st_v.at[pl.ds(0, _DUM)],
                                    acc_sh.at[pl.ds(_CH, _DUM)])
            plsc.subcore_barrier()

            def win(k, _):
                off = (s + 16 * k) * _WG
                pltpu.sync_copy(didx_hbm.at[pl.ds(off, _WG)], didx_v)
                _localize_dst(didx_v, lo)
                if gather_rows:
                    pltpu.sync_copy(sidx_hbm.at[pl.ds(off, _WG)], sidx_v)
                    pltpu.async_copy(
                        rows_hbm.at[sidx_v], rows_v, sem).wait()
                else:
                    pltpu.sync_copy(rows_hbm.at[pl.ds(off, _WG)], rows_v)
                pltpu.sync_copy(rows_v, acc_sh.at[didx_v], add=True)
                return 0
            lax.fori_loop(0, n_w, win, 0)
            plsc.subcore_barrier()
            pltpu.sync_copy(acc_sh.at[pl.ds(s * tpr, tpr)], st_v)
            pltpu.sync_copy(st_v, out_hbm.at[pl.ds(lo + s * tpr, tpr)])
            if not gather_rows:
                _zero_vmem(st_v, tpr)
            return 0
        lax.fori_loop(0, _NCHUNK // 2, one_pass, 0)

    if gather_rows:
        return make

    def body(didx_hbm, rows_hbm, out_hbm, didx_v, rows_v, st_v, acc_sh, sem):
        return make(None, didx_hbm, rows_hbm, None, out_hbm,
                    None, didx_v, rows_v, st_v, acc_sh, sem)
    return body


_cb_call = pl.kernel(
    _chunked_scatter_body(False),
    out_type=jax.ShapeDtypeStruct((_EPAD, _D), jnp.float32),
    mesh=_mesh,
    scratch_types=[
        pltpu.VMEM((_WG,), jnp.int32),
        pltpu.VMEM((_WG, _D), jnp.float32),
        pltpu.VMEM((_CH // 16, _D), jnp.float32),
        pltpu.VMEM_SHARED((_CH + _DUM, _D), jnp.float32),
        pltpu.SemaphoreType.DMA,
    ],
    compiler_params=pltpu.CompilerParams(use_tc_tiling_on_sc=False),
)

_aggb_call = pl.kernel(
    _chunked_scatter_body(True),
    out_type=jax.ShapeDtypeStruct((_EPAD, _D), jnp.float32),
    mesh=_mesh,
    scratch_types=[
        pltpu.VMEM((_WG,), jnp.int32),
        pltpu.VMEM((_WG,), jnp.int32),
        pltpu.VMEM((_WG, _D), jnp.float32),
        pltpu.VMEM((_CH // 16, _D), jnp.float32),
        pltpu.VMEM_SHARED((_CH + _DUM, _D), jnp.float32),
        pltpu.SemaphoreType.DMA,
    ],
    compiler_params=pltpu.CompilerParams(use_tc_tiling_on_sc=False),
)


def _agga_body(sidx_hbm, didx_hbm, node_hbm, bond_hbm, out_hbm,
               sidx_v, didx_v, rows_v, brows_v, st_v, ntab_sh, acc_sh, sem):
    c = lax.axis_index("c")
    s = lax.axis_index("s")
    wid = s * 2 + c
    tpr = 624
    pltpu.sync_copy(node_hbm.at[pl.ds(s * tpr, tpr)], st_v)
    pltpu.sync_copy(st_v, ntab_sh.at[pl.ds(s * tpr, tpr)])

    @pl.when(s == 0)
    def _():
        pltpu.sync_copy(node_hbm.at[pl.ds(16 * tpr, 16)],
                        st_v.at[pl.ds(0, 16)])
        pltpu.sync_copy(st_v.at[pl.ds(0, 16)],
                        ntab_sh.at[pl.ds(16 * tpr, 16)])
    _zero_vmem(st_v, tpr)
    pltpu.sync_copy(st_v, acc_sh.at[pl.ds(s * tpr, tpr)])

    @pl.when(s == 0)
    def _():
        pltpu.sync_copy(st_v.at[pl.ds(0, 16)],
                        acc_sh.at[pl.ds(16 * tpr, 16)])
    plsc.subcore_barrier()

    n_w = jnp.where(wid < _NW_E % 32, _NW_E // 32 + 1, _NW_E // 32)

    def win(k, _):
        off = (wid + 32 * k) * _WG
        pltpu.sync_copy(sidx_hbm.at[pl.ds(off, _WG)], sidx_v)
        pltpu.sync_copy(didx_hbm.at[pl.ds(off, _WG)], didx_v)
        pltpu.async_copy(ntab_sh.at[sidx_v], rows_v, sem).wait()
        pltpu.sync_copy(bond_hbm.at[pl.ds(off, _WG)], brows_v)

        def add(i, _):
            rows_v[i, pl.ds(0, 16)] = (
                rows_v[i, pl.ds(0, 16)] + brows_v[i, pl.ds(0, 16)])
            rows_v[i, pl.ds(16, 16)] = (
                rows_v[i, pl.ds(16, 16)] + brows_v[i, pl.ds(16, 16)])
            return 0
        lax.fori_loop(0, _WG, add, 0)
        pltpu.sync_copy(rows_v, acc_sh.at[didx_v], add=True)
        return 0
    lax.fori_loop(0, n_w, win, 0)
    plsc.subcore_barrier()
    pltpu.sync_copy(acc_sh.at[pl.ds(s * tpr, tpr)], st_v)
    pltpu.sync_copy(st_v, out_hbm.at[c, pl.ds(s * tpr, tpr)])

    @pl.when(s == 0)
    def _():
        pltpu.sync_copy(acc_sh.at[pl.ds(16 * tpr, 16)],
                        st_v.at[pl.ds(0, 16)])
        pltpu.sync_copy(st_v.at[pl.ds(0, 16)],
                        out_hbm.at[c, pl.ds(16 * tpr, 16)])


_agga_call = pl.kernel(
    _agga_body,
    out_type=jax.ShapeDtypeStruct((2, _N, _D), jnp.float32),
    mesh=_mesh,
    scratch_types=[
        pltpu.VMEM((_WG,), jnp.int32),
        pltpu.VMEM((_WG,), jnp.int32),
        pltpu.VMEM((_WG, _D), jnp.float32),
        pltpu.VMEM((_WG, _D), jnp.float32),
        pltpu.VMEM((624, _D), jnp.float32),
        pltpu.VMEM_SHARED((_N, _D), jnp.float32),
        pltpu.VMEM_SHARED((_N, _D), jnp.float32),
        pltpu.SemaphoreType.DMA,
    ],
    compiler_params=pltpu.CompilerParams(use_tc_tiling_on_sc=False),
)


def _bond_mm_body(x_ref, g_ref, w_ref, b_ref, o_ref):
    acc = x_ref[...] + g_ref[...]
    o_ref[...] = jnp.maximum(
        jnp.dot(acc, w_ref[...], preferred_element_type=jnp.float32)
        + b_ref[...], 0.0)


def _bond_mm(x, g, w, b):
    return pl.pallas_call(
        _bond_mm_body,
        grid=(_E // _WG,),
        in_specs=[
            pl.BlockSpec((_WG, _D), lambda i: (i, 0)),
            pl.BlockSpec((_WG, _D), lambda i: (i, 0)),
            pl.BlockSpec((_D, _D), lambda i: (0, 0)),
            pl.BlockSpec((1, _D), lambda i: (0, 0)),
        ],
        out_specs=pl.BlockSpec((_WG, _D), lambda i: (i, 0)),
        out_shape=jax.ShapeDtypeStruct((_E, _D), jnp.float32),
    )(x, g, w, b)


def _node_mm_body(x_ref, g_ref, w_ref, b_ref, o_ref):
    acc = x_ref[...] + g_ref[0] + g_ref[1]
    o_ref[...] = jnp.maximum(
        jnp.dot(acc, w_ref[...], preferred_element_type=jnp.float32)
        + b_ref[...], 0.0)


def _node_mm(x, g, w, b):
    return pl.pallas_call(
        _node_mm_body,
        out_shape=jax.ShapeDtypeStruct((_N, _D), jnp.float32),
    )(x, g, w, b)


def _head_body(n3_ref, m_ref, gm_ref, bt_ref, w1_ref, b1_ref, w2_ref, b2_ref,
               o_ref):
    sign = 1.0 - 2.0 * m_ref[...]
    diff = jnp.sum(n3_ref[...] * sign[:, :, None], axis=1)
    mean = jnp.mean(diff, axis=-1, keepdims=True)
    ctr = diff - mean
    var = jnp.mean(ctr * ctr, axis=-1, keepdims=True)
    normed = ctr * lax.rsqrt(var + 1e-5) * gm_ref[...] + bt_ref[...]
    h = jnp.dot(normed, w1_ref[...], preferred_element_type=jnp.float32)
    h = h + b1_ref[...]
    h = jnp.where(h > 0, h, 0.01 * h)
    o_ref[...] = (
        jnp.dot(h, w2_ref[...], preferred_element_type=jnp.float32)
        + b2_ref[...])


def _head(node, maskf, gamma, beta, w1, b1, w2, b2):
    n3 = node.reshape(_B, _NPG, _D)
    return pl.pallas_call(
        _head_body,
        out_shape=jax.ShapeDtypeStruct((_B, 1), jnp.float32),
    )(n3, maskf, gamma.reshape(1, _D), beta.reshape(1, _D),
      w1, b1.reshape(1, _H), w2, b2.reshape(1, 1))


def kernel(x, edge_index, edge_attr, angle_edge_index, angle_attr,
           is_reactant, Wa, ba, Wb, bb, gamma, beta, W1, b1, W2, b2):
    src, dst = edge_index[0], edge_index[1]
    asrc, adst = angle_edge_index[0], angle_edge_index[1]
    maskf = is_reactant.astype(jnp.float32).reshape(_B, _NPG)

    cb = _cb_call(adst, angle_attr)

    def layer(l, carry):
        node, bond = carry
        wa = lax.dynamic_index_in_dim(Wa, l, keepdims=False)
        baa = lax.dynamic_index_in_dim(ba, l, keepdims=False)
        wb = lax.dynamic_index_in_dim(Wb, l, keepdims=False)
        bbb = lax.dynamic_index_in_dim(bb, l, keepdims=False)
        aggb = _aggb_call(asrc, adst, bond, cb)
        bond = _bond_mm(bond, aggb, wb, bbb.reshape(1, _D))
        agga = _agga_call(src, dst, node, bond)
        node = _node_mm(node, agga, wa, baa.reshape(1, _D))
        return node, bond

    node, bond = lax.fori_loop(0, _L, layer, (x, edge_attr))
    return _head(node, maskf, gamma, beta, W1, b1, W2, b2)
